# baseline (speedup 1.0000x reference)
import functools

import numpy as np
import jax
import jax.numpy as jnp
from jax import lax
from jax.experimental import pallas as pl
from jax.experimental.pallas import tpu as pltpu
from jax.experimental.pallas import tpu_sc as plsc

_B, _P, _F = 4096, 10, 7
_MIN_SCORE = 0.3

_INFO = plsc.get_sparse_core_info()
_NC, _NS, _L = _INFO.num_cores, _INFO.num_subcores, _INFO.num_lanes
_NW = _NC * _NS
_RW = _B // _NW
_IDX_W = _RW * _P * 3
_CONF_W = _RW * _P
_BBOX_W = _RW * _P * 2
_OUT_W = _RW * _P * _F
_MASK_BASE = _IDX_W
_CONF_BASE = _IDX_W + _CONF_W
_BBOX_BASE = _IDX_W + 2 * _CONF_W
_PERIOD = 8 * _P * _F
_NBLK = _OUT_W // _PERIOD


def _build_tables():
    a = np.zeros(_PERIOD, np.int32)
    s = np.zeros(_PERIOD, np.int32)
    for j in range(_PERIOD):
        row, k = divmod(j, _P * _F)
        p, f = divmod(k, _F)
        if f < 3:
            a[j] = row * (_P * 3) + p * 3 + f
            s[j] = 8 * _P * 3
        elif f == 3:
            a[j] = _MASK_BASE + row * _P + p
            s[j] = 8 * _P
        elif f == 4:
            a[j] = _CONF_BASE + row * _P + p
            s[j] = 8 * _P
        else:
            a[j] = _BBOX_BASE + row * (_P * 2) + p * 2 + (f - 5)
            s[j] = 8 * _P * 2
    return a, s


_ATAB_NP, _STAB_NP = _build_tables()


@functools.partial(
    pl.kernel,
    mesh=plsc.VectorSubcoreMesh(core_axis_name="c", subcore_axis_name="s"),
    out_type=jax.ShapeDtypeStruct((_B * _P * _F,), jnp.float32),
    compiler_params=pltpu.CompilerParams(
        needs_layout_passes=False, skip_device_barrier=True),
    scratch_types=[
        pltpu.VMEM((_OUT_W,), jnp.float32),
        pltpu.VMEM((_OUT_W,), jnp.float32),
        pltpu.VMEM((_PERIOD,), jnp.int32),
        pltpu.VMEM((_PERIOD,), jnp.int32),
    ],
)
def _sc_assemble(idx_hbm, conf_hbm, bbox_hbm, atab_hbm, stab_hbm, out_hbm,
                 stage, outb, atab, stab):
    wid = lax.axis_index("s") * _NC + lax.axis_index("c")
    pltpu.sync_copy(idx_hbm.at[pl.ds(wid * _IDX_W, _IDX_W)],
                    stage.at[pl.ds(0, _IDX_W)])
    pltpu.sync_copy(conf_hbm.at[pl.ds(wid * _CONF_W, _CONF_W)],
                    stage.at[pl.ds(_CONF_BASE, _CONF_W)])
    pltpu.sync_copy(bbox_hbm.at[pl.ds(wid * _BBOX_W, _BBOX_W)],
                    stage.at[pl.ds(_BBOX_BASE, _BBOX_W)])
    pltpu.sync_copy(atab_hbm, atab)
    pltpu.sync_copy(stab_hbm, stab)

    def mask_step(i, carry):
        c = stage[pl.ds(_CONF_BASE + i * _L, _L)]
        stage[pl.ds(_MASK_BASE + i * _L, _L)] = jnp.where(
            c > _MIN_SCORE, jnp.float32(0.0), jnp.float32(-1.0))
        return carry

    lax.fori_loop(0, _CONF_W // _L, mask_step, 0)

    def blk_step(b, carry):
        boff = b * _PERIOD
        for t in range(_PERIOD // _L):
            a = atab[pl.ds(t * _L, _L)]
            s = stab[pl.ds(t * _L, _L)]
            g = a + s * b
            outb[pl.ds(boff + t * _L, _L)] = plsc.load_gather(stage, [g])
        return carry

    lax.fori_loop(0, _NBLK, blk_step, 0)
    pltpu.sync_copy(outb, out_hbm.at[pl.ds(wid * _OUT_W, _OUT_W)])


def kernel(topk_index, topk_confs, match_bbox_preds, meta):
    del meta
    out = _sc_assemble(
        topk_index.reshape(-1),
        topk_confs.reshape(-1),
        match_bbox_preds.reshape(-1),
        jnp.asarray(_ATAB_NP),
        jnp.asarray(_STAB_NP),
    )
    return out.reshape(_B, _P, _F)


# speedup vs baseline: 1.0802x; 1.0802x over previous
import functools

import numpy as np
import jax
import jax.numpy as jnp
from jax import lax
from jax.experimental import pallas as pl
from jax.experimental.pallas import tpu as pltpu
from jax.experimental.pallas import tpu_sc as plsc

_B, _P, _F = 4096, 10, 7
_MIN_SCORE = 0.3

_INFO = plsc.get_sparse_core_info()
_NC, _NS, _L = _INFO.num_cores, _INFO.num_subcores, _INFO.num_lanes
_NW = _NC * _NS
_RW = _B // _NW
_CB = 16
_NCH = _RW // _CB
_NREC = _CB * _P

_NI = _NREC * 3
_NX = _NREC * 2
_NCF = _NREC


def _rc_tables(n_feat):
    j = np.arange(_NREC * n_feat, dtype=np.int32)
    return j // n_feat, j % n_feat


_IRT, _ICT = _rc_tables(3)
_XRT, _XCT = _rc_tables(2)
_CRT, _ = _rc_tables(1)
_CBT, _CPT = _CRT // _P, _CRT % _P


@functools.partial(
    pl.kernel,
    mesh=plsc.VectorSubcoreMesh(core_axis_name="c", subcore_axis_name="s"),
    out_type=jax.ShapeDtypeStruct((_B, _P, _F), jnp.float32),
    compiler_params=pltpu.CompilerParams(needs_layout_passes=False),
    scratch_types=[
        pltpu.VMEM((_NREC, 3), jnp.float32),
        pltpu.VMEM((_CB, _P), jnp.float32),
        pltpu.VMEM((_NREC, 2), jnp.float32),
        pltpu.VMEM((_NREC, _F), jnp.float32),
        pltpu.VMEM((_NI,), jnp.int32),
        pltpu.VMEM((_NI,), jnp.int32),
        pltpu.VMEM((_NX,), jnp.int32),
        pltpu.VMEM((_NX,), jnp.int32),
        pltpu.VMEM((_NCF,), jnp.int32),
        pltpu.VMEM((_NCF,), jnp.int32),
    ],
)
def _sc_assemble(idx_hbm, conf_hbm, bbox_hbm,
                 ir_hbm, ic_hbm, xr_hbm, xc_hbm, cb_hbm, cp_hbm, out_hbm,
                 idx_v, conf_v, bbox_v, out_v, ir, ic, xr, xc, cb, cp):
    wid = lax.axis_index("s") * _NC + lax.axis_index("c")
    pltpu.sync_copy(ir_hbm, ir)
    pltpu.sync_copy(ic_hbm, ic)
    pltpu.sync_copy(xr_hbm, xr)
    pltpu.sync_copy(xc_hbm, xc)
    pltpu.sync_copy(cb_hbm, cb)
    pltpu.sync_copy(cp_hbm, cp)

    five = jnp.full((_L,), 5, jnp.int32)
    three = jnp.full((_L,), 3, jnp.int32)
    four = jnp.full((_L,), 4, jnp.int32)

    def chunk_step(ch, carry):
        rows = pl.ds(wid * _RW + ch * _CB, _CB)
        pltpu.sync_copy(idx_hbm.at[rows], idx_v.reshape(_CB, _P, 3))
        pltpu.sync_copy(conf_hbm.at[rows], conf_v)
        pltpu.sync_copy(bbox_hbm.at[rows], bbox_v.reshape(_CB, _P, 2))
        for t in range(_NI // _L):
            o = pl.ds(t * _L, _L)
            r = ir[o]
            c = ic[o]
            plsc.store_scatter(out_v, [r, c],
                               plsc.load_gather(idx_v, [r, c]))
        for t in range(_NX // _L):
            o = pl.ds(t * _L, _L)
            r = xr[o]
            c = xc[o]
            plsc.store_scatter(out_v, [r, c + five],
                               plsc.load_gather(bbox_v, [r, c]))
        for t in range(_NCF // _L):
            o = pl.ds(t * _L, _L)
            b = cb[o]
            p = cp[o]
            r = b * _P + p
            cvals = plsc.load_gather(conf_v, [b, p])
            m = jnp.where(cvals > _MIN_SCORE, jnp.float32(0.0),
                          jnp.float32(-1.0))
            plsc.store_scatter(out_v, [r, four], cvals)
            plsc.store_scatter(out_v, [r, three], m)
        pltpu.sync_copy(out_v.reshape(_CB, _P, _F), out_hbm.at[rows])
        return carry

    lax.fori_loop(0, _NCH, chunk_step, 0)


def kernel(topk_index, topk_confs, match_bbox_preds, meta):
    del meta
    return _sc_assemble(
        topk_index, topk_confs, match_bbox_preds,
        jnp.asarray(_IRT), jnp.asarray(_ICT),
        jnp.asarray(_XRT), jnp.asarray(_XCT),
        jnp.asarray(_CBT), jnp.asarray(_CPT),
    )


# speedup vs baseline: 1.4083x; 1.3037x over previous
import functools

import jax
import jax.numpy as jnp
from jax.experimental import pallas as pl
from jax.experimental.pallas import tpu as pltpu

_B, _P, _F = 4096, 10, 7
_MIN_SCORE = 0.3
_BN = 512


def _body(idx_ref, conf_ref, bbox_ref, out_ref):
    idx = idx_ref[...]
    conf = conf_ref[...]
    bbox = bbox_ref[...]
    mask = jnp.where(conf > _MIN_SCORE, jnp.float32(0.0), jnp.float32(-1.0))
    out_ref[...] = jnp.concatenate(
        [idx, mask[..., None], conf[..., None], bbox], axis=-1)


@jax.jit
def _tc_assemble(topk_index, topk_confs, match_bbox_preds):
    grid = (_B // _BN,)
    return pl.pallas_call(
        _body,
        grid=grid,
        in_specs=[
            pl.BlockSpec((_BN, _P, 3), lambda i: (i, 0, 0)),
            pl.BlockSpec((_BN, _P), lambda i: (i, 0)),
            pl.BlockSpec((_BN, _P, 2), lambda i: (i, 0, 0)),
        ],
        out_specs=pl.BlockSpec((_BN, _P, _F), lambda i: (i, 0, 0)),
        out_shape=jax.ShapeDtypeStruct((_B, _P, _F), jnp.float32),
    )(topk_index, topk_confs, match_bbox_preds)


def kernel(topk_index, topk_confs, match_bbox_preds, meta):
    del meta
    return _tc_assemble(topk_index, topk_confs, match_bbox_preds)
